# Initial kernel scaffold; baseline (speedup 1.0000x reference)
import jax
import jax.numpy as jnp
from jax.experimental import pallas as pl


def kernel(z, embedding):
    raise NotImplementedError("write your pallas kernel here")


# speedup vs baseline: 2.1041x; 2.1041x over previous
import functools

import jax
import jax.numpy as jnp
from jax import lax
from jax.experimental import pallas as pl
from jax.experimental.pallas import tpu as pltpu
from jax.experimental.pallas import tpu_sc as plsc

_N_E = 8192
_E_DIM = 64
_B = 4
_S = 32 * 32 * 8
_T = _B * _S

_TC = 512
_NB = _S // _TC
_G = _T // _TC

_NC = 2
_NS = 16
_NW = _NC * _NS
_BPW = _T // _NW


def _argmin_body(z_ref, emb_ref, idx_ref):
    e = emb_ref[...]
    zb = z_ref[0]
    e2 = jnp.sum(e * e, axis=1, keepdims=True)
    prod = lax.dot_general(
        e.astype(jnp.bfloat16), zb.astype(jnp.bfloat16),
        (((1,), (0,)), ((), ())),
        preferred_element_type=jnp.float32,
    )
    scores = e2 - 2.0 * prod
    idx_ref[0, 0] = jnp.argmin(scores, axis=0).astype(jnp.int32)


def _tc_argmin(zt, embedding):
    return pl.pallas_call(
        _argmin_body,
        grid=(_G,),
        in_specs=[
            pl.BlockSpec((1, _E_DIM, _TC), lambda g: (g // _NB, 0, g % _NB)),
            pl.BlockSpec((_N_E, _E_DIM), lambda g: (0, 0)),
        ],
        out_specs=pl.BlockSpec((1, 1, _TC), lambda g: (g, 0, 0)),
        out_shape=jax.ShapeDtypeStruct((_G, 1, _TC), jnp.int32),
        compiler_params=pltpu.CompilerParams(
            dimension_semantics=("arbitrary",),
        ),
    )(zt, embedding)


_D_PAD = 128
_NSUB = 2
_CH = _BPW // _NSUB


@functools.cache
def _sc_gather_kernel():
    @functools.partial(
        pl.kernel,
        mesh=plsc.VectorSubcoreMesh(core_axis_name="c", subcore_axis_name="s"),
        out_type=jax.ShapeDtypeStruct((_T, _D_PAD), jnp.float32),
        scratch_types=[
            pltpu.VMEM((_CH,), jnp.int32),
            pltpu.VMEM((_CH, _D_PAD), jnp.float32),
            pltpu.SemaphoreType.DMA,
        ],
    )
    def _sc_gather(table_hbm, idx_hbm, out_hbm, idx_v, rows_v, sem):
        wid = lax.axis_index("s") * _NC + lax.axis_index("c")
        base = wid * _BPW
        for j in range(_NSUB):
            pltpu.sync_copy(idx_hbm.at[pl.ds(base + j * _CH, _CH)], idx_v)
            pltpu.async_copy(table_hbm.at[idx_v], rows_v, sem).wait()
            pltpu.sync_copy(rows_v, out_hbm.at[pl.ds(base + j * _CH, _CH)])

    return _sc_gather


def kernel(z, embedding):
    zt = z.reshape(_B, _E_DIM, _S)
    idx = _tc_argmin(zt, embedding).reshape(-1)
    table = jnp.pad(embedding, ((0, 0), (0, _D_PAD - _E_DIM)))
    zq_flat = _sc_gather_kernel()(table, idx)[:, :_E_DIM]
    z_q = zq_flat.reshape(_B, _S, _E_DIM).transpose(0, 2, 1)
    z_q = z_q.reshape(z.shape)
    zero = jnp.array(0.0, dtype=jnp.float32)
    return (z_q, (zero, zero, zero, zero), idx)


# speedup vs baseline: 2.5505x; 1.2122x over previous
import functools

import jax
import jax.numpy as jnp
from jax import lax
from jax.experimental import pallas as pl
from jax.experimental.pallas import tpu as pltpu
from jax.experimental.pallas import tpu_sc as plsc

_N_E = 8192
_E_DIM = 64
_B = 4
_S = 32 * 32 * 8
_T = _B * _S

_TC = 512
_NB = _S // _TC
_G = _T // _TC

_NC = 2
_NS = 16
_NW = _NC * _NS
_BPW = _T // _NW


def _argmin_body(z_ref, emb_ref, idx_ref, eneg_ref, e2_ref):
    @pl.when(pl.program_id(0) == 0)
    def _():
        e = emb_ref[...]
        eneg_ref[...] = (-2.0 * e).astype(jnp.bfloat16)
        e2_ref[...] = jnp.sum(e * e, axis=1, keepdims=True)

    zb = z_ref[0].astype(jnp.bfloat16)
    scores = e2_ref[...] + lax.dot_general(
        eneg_ref[...], zb, (((1,), (0,)), ((), ())),
        preferred_element_type=jnp.float32,
    )
    idx_ref[0, 0] = jnp.argmin(scores, axis=0).astype(jnp.int32)


def _tc_argmin(zt, embedding):
    return pl.pallas_call(
        _argmin_body,
        grid=(_G,),
        in_specs=[
            pl.BlockSpec((1, _E_DIM, _TC), lambda g: (g // _NB, 0, g % _NB)),
            pl.BlockSpec((_N_E, _E_DIM), lambda g: (0, 0)),
        ],
        out_specs=pl.BlockSpec((1, 1, _TC), lambda g: (g, 0, 0)),
        out_shape=jax.ShapeDtypeStruct((_G, 1, _TC), jnp.int32),
        scratch_shapes=[pltpu.VMEM((_N_E, _E_DIM), jnp.bfloat16),
                        pltpu.VMEM((_N_E, 1), jnp.float32)],
        compiler_params=pltpu.CompilerParams(
            dimension_semantics=("arbitrary",),
        ),
    )(zt, embedding)


_D_PAD = 128
_NSUB = 2
_CH = _BPW // _NSUB


@functools.cache
def _sc_gather_kernel():
    @functools.partial(
        pl.kernel,
        mesh=plsc.VectorSubcoreMesh(core_axis_name="c", subcore_axis_name="s"),
        out_type=jax.ShapeDtypeStruct((_T, _D_PAD), jnp.float32),
        scratch_types=[
            pltpu.VMEM((_CH,), jnp.int32),
            pltpu.VMEM((_CH, _D_PAD), jnp.float32),
            pltpu.SemaphoreType.DMA,
        ],
    )
    def _sc_gather(table_hbm, idx_hbm, out_hbm, idx_v, rows_v, sem):
        wid = lax.axis_index("s") * _NC + lax.axis_index("c")
        base = wid * _BPW
        for j in range(_NSUB):
            pltpu.sync_copy(idx_hbm.at[pl.ds(base + j * _CH, _CH)], idx_v)
            pltpu.async_copy(table_hbm.at[idx_v], rows_v, sem).wait()
            pltpu.sync_copy(rows_v, out_hbm.at[pl.ds(base + j * _CH, _CH)])

    return _sc_gather


def kernel(z, embedding):
    zt = z.reshape(_B, _E_DIM, _S)
    idx = _tc_argmin(zt, embedding).reshape(-1)
    table = jnp.pad(embedding, ((0, 0), (0, _D_PAD - _E_DIM)))
    zq_flat = _sc_gather_kernel()(table, idx)[:, :_E_DIM]
    z_q = zq_flat.reshape(_B, _S, _E_DIM).transpose(0, 2, 1)
    z_q = z_q.reshape(z.shape)
    zero = jnp.array(0.0, dtype=jnp.float32)
    return (z_q, (zero, zero, zero, zero), idx)


# speedup vs baseline: 2.7003x; 1.0588x over previous
import functools

import jax
import jax.numpy as jnp
from jax import lax
from jax.experimental import pallas as pl
from jax.experimental.pallas import tpu as pltpu
from jax.experimental.pallas import tpu_sc as plsc

_N_E = 8192
_E_DIM = 64
_B = 4
_S = 32 * 32 * 8
_T = _B * _S

_TC = 512
_NB = _S // _TC
_G = _T // _TC

_NC = 2
_NS = 16
_NW = _NC * _NS
_BPW = _T // _NW


_K_AUG = 80


def _argmin_body(z_ref, emb_ref, idx_ref, eaug_ref):
    @pl.when(pl.program_id(0) == 0)
    def _():
        e = emb_ref[...]
        e2 = jnp.sum(e * e, axis=1, keepdims=True)
        hi = e2.astype(jnp.bfloat16)
        r1 = e2 - hi.astype(jnp.float32)
        lo = r1.astype(jnp.bfloat16)
        lo2 = (r1 - lo.astype(jnp.float32)).astype(jnp.bfloat16)
        zpad = jnp.zeros((_N_E, _K_AUG - _E_DIM - 3), jnp.bfloat16)
        eaug_ref[...] = jnp.concatenate(
            [(-2.0 * e).astype(jnp.bfloat16), hi, lo, lo2, zpad], axis=1)

    zb = z_ref[0].astype(jnp.bfloat16)
    zaug = jnp.concatenate(
        [zb, jnp.ones((_K_AUG - _E_DIM, _TC), jnp.bfloat16)], axis=0)
    scores = lax.dot_general(
        eaug_ref[...], zaug, (((1,), (0,)), ((), ())),
        preferred_element_type=jnp.float32,
    )
    idx_ref[0, 0] = jnp.argmin(scores, axis=0).astype(jnp.int32)


def _tc_argmin(zt, embedding):
    return pl.pallas_call(
        _argmin_body,
        grid=(_G,),
        in_specs=[
            pl.BlockSpec((1, _E_DIM, _TC), lambda g: (g // _NB, 0, g % _NB)),
            pl.BlockSpec((_N_E, _E_DIM), lambda g: (0, 0)),
        ],
        out_specs=pl.BlockSpec((1, 1, _TC), lambda g: (g, 0, 0)),
        out_shape=jax.ShapeDtypeStruct((_G, 1, _TC), jnp.int32),
        scratch_shapes=[pltpu.VMEM((_N_E, _K_AUG), jnp.bfloat16)],
        compiler_params=pltpu.CompilerParams(
            dimension_semantics=("arbitrary",),
        ),
    )(zt, embedding)


_D_PAD = 128
_NSUB = 2
_CH = _BPW // _NSUB


@functools.cache
def _sc_gather_kernel():
    @functools.partial(
        pl.kernel,
        mesh=plsc.VectorSubcoreMesh(core_axis_name="c", subcore_axis_name="s"),
        out_type=jax.ShapeDtypeStruct((_T, _D_PAD), jnp.float32),
        scratch_types=[
            pltpu.VMEM((_CH,), jnp.int32),
            pltpu.VMEM((_CH, _D_PAD), jnp.float32),
            pltpu.SemaphoreType.DMA,
        ],
    )
    def _sc_gather(table_hbm, idx_hbm, out_hbm, idx_v, rows_v, sem):
        wid = lax.axis_index("s") * _NC + lax.axis_index("c")
        base = wid * _BPW
        for j in range(_NSUB):
            pltpu.sync_copy(idx_hbm.at[pl.ds(base + j * _CH, _CH)], idx_v)
            pltpu.async_copy(table_hbm.at[idx_v], rows_v, sem).wait()
            pltpu.sync_copy(rows_v, out_hbm.at[pl.ds(base + j * _CH, _CH)])

    return _sc_gather


def kernel(z, embedding):
    zt = z.reshape(_B, _E_DIM, _S)
    idx = _tc_argmin(zt, embedding).reshape(-1)
    table = jnp.pad(embedding, ((0, 0), (0, _D_PAD - _E_DIM)))
    zq_flat = _sc_gather_kernel()(table, idx)[:, :_E_DIM]
    z_q = zq_flat.reshape(_B, _S, _E_DIM).transpose(0, 2, 1)
    z_q = z_q.reshape(z.shape)
    zero = jnp.array(0.0, dtype=jnp.float32)
    return (z_q, (zero, zero, zero, zero), idx)
